# baseline (speedup 1.0000x reference)
import functools
import math

import jax
import jax.numpy as jnp
import numpy as np
from jax.experimental import pallas as pl
from jax.experimental.pallas import tpu as pltpu

_STRIDES = (8.0, 16.0, 32.0)
_ANCHORS = np.array(
    [[[10, 13], [16, 30], [33, 23]],
     [[30, 61], [62, 45], [59, 119]],
     [[116, 90], [156, 198], [373, 326]]], dtype=np.float32)
_SCORE_T = 0.25
_NMS_T = 0.45
_DET = 300
_KPRE = 1000
_MAXSZ = 4096.0
_IMH = 640.0
_IMW = 640.0
_NC = 80


_N_BOX = 25200
_LVL_N = (19200, 4800, 1200)
_LVL_DIM = (80, 40, 20)
_R_CHUNK = 3600


def _rowmeta(r):
    rf = r
    in1 = rf >= _LVL_N[0]
    in2 = rf >= _LVL_N[0] + _LVL_N[1]
    base = jnp.where(in2, _LVL_N[0] + _LVL_N[1], jnp.where(in1, _LVL_N[0], 0))
    dim = jnp.where(in2, _LVL_DIM[2], jnp.where(in1, _LVL_DIM[1], _LVL_DIM[0]))
    rl = rf - base
    cell = rl // 3
    a = rl - cell * 3
    gx = (cell % dim).astype(jnp.float32)
    gy = (cell // dim).astype(jnp.float32)
    stride = jnp.where(in2, 32.0, jnp.where(in1, 16.0, 8.0))
    aw = jnp.zeros_like(gx)
    ah = jnp.zeros_like(gx)
    for lv in range(3):
        if lv == 0:
            msk = ~in1
        elif lv == 1:
            msk = in1 & ~in2
        else:
            msk = in2
        for ai in range(3):
            sel = msk & (a == ai)
            aw = jnp.where(sel, _ANCHORS[lv, ai, 0], aw)
            ah = jnp.where(sel, _ANCHORS[lv, ai, 1], ah)
    return gx, gy, aw, ah, stride


def _decode_kern(p_ref, boxes_ref, logits_ref):
    c = pl.program_id(1)
    p = jax.nn.sigmoid(p_ref[0])
    R = p.shape[0]
    r = c * _R_CHUNK + jax.lax.broadcasted_iota(jnp.int32, (R, 1), 0)
    gx, gy, aw, ah, stride = _rowmeta(r)
    x = (2.0 * p[:, 0:1] - 0.5 + gx) * stride
    y = (2.0 * p[:, 1:2] - 0.5 + gy) * stride
    w = 4.0 * jnp.square(p[:, 2:3]) * aw
    h = 4.0 * jnp.square(p[:, 3:4]) * ah
    obj = p[:, 4:5]
    cls = p[:, 5:85]
    gate = (obj > _SCORE_T).astype(p.dtype)
    boxes_ref[0] = jnp.concatenate([x, y, w, h], axis=-1)
    logits_ref[0] = obj * cls * gate


def _decode(pred0, pred1, pred2):
    B = pred0.shape[0]
    p = jnp.concatenate([pred0.reshape(B, -1, 85), pred1.reshape(B, -1, 85),
                         pred2.reshape(B, -1, 85)], axis=1)
    n = _N_BOX
    nc = n // _R_CHUNK
    boxes, logits = pl.pallas_call(
        _decode_kern,
        grid=(B, nc),
        in_specs=[
            pl.BlockSpec((1, _R_CHUNK, 85), lambda b, c: (b, c, 0)),
        ],
        out_specs=[
            pl.BlockSpec((1, _R_CHUNK, 4), lambda b, c: (b, c, 0)),
            pl.BlockSpec((1, _R_CHUNK, _NC), lambda b, c: (b, c, 0)),
        ],
        out_shape=[
            jax.ShapeDtypeStruct((B, n, 4), jnp.float32),
            jax.ShapeDtypeStruct((B, n, _NC), jnp.float32),
        ],
    )(p)
    return boxes, logits


def _box_iou(a, b):
    lt = jnp.maximum(a[:, None, :2], b[None, :, :2])
    rb = jnp.minimum(a[:, None, 2:], b[None, :, 2:])
    wh = jnp.clip(rb - lt, 0.0, None)
    inter = wh[..., 0] * wh[..., 1]
    area_a = (a[:, 2] - a[:, 0]) * (a[:, 3] - a[:, 1])
    area_b = (b[:, 2] - b[:, 0]) * (b[:, 3] - b[:, 1])
    return inter / (area_a[:, None] + area_b[None, :] - inter + 1e-7)


def _per_image(boxes_i, logits_i, scale_i):
    flat = logits_i.reshape(-1)
    s = jnp.where(flat > _SCORE_T, flat, -1.0)
    sc, idx = jax.lax.top_k(s, _KPRE)
    bi = idx // _NC
    lab = idx % _NC
    bx = boxes_i[bi]
    x1 = jnp.clip(bx[:, 0] - bx[:, 2] * 0.5, 0.0, _IMW)
    y1 = jnp.clip(bx[:, 1] - bx[:, 3] * 0.5, 0.0, _IMH)
    x2 = jnp.clip(bx[:, 0] + bx[:, 2] * 0.5, 0.0, _IMW)
    y2 = jnp.clip(bx[:, 1] + bx[:, 3] * 0.5, 0.0, _IMH)
    bxy = jnp.stack([x1, y1, x2, y2], axis=1)
    off = lab.astype(jnp.float32)[:, None] * _MAXSZ
    nb = bxy + off
    iou = _box_iou(nb, nb)
    keep0 = sc > _SCORE_T

    def body(i, keep):
        sup = (iou[i] > _NMS_T) & (jnp.arange(_KPRE) > i) & keep[i]
        return keep & (~sup)

    keep = jax.lax.fori_loop(0, _KPRE, body, keep0)
    ks = jnp.where(keep, sc, -1.0)
    fs, fi = jax.lax.top_k(ks, _DET)
    fb = bxy[fi] / scale_i
    fl = lab[fi].astype(jnp.float32)
    m = (fs > _SCORE_T).astype(jnp.float32)
    return jnp.concatenate([fb * m[:, None], (fs * m)[:, None], (fl * m)[:, None]], axis=1)


def kernel(pred0, pred1, pred2, scale_factors):
    boxes, logits = _decode(pred0, pred1, pred2)
    return jax.vmap(_per_image)(boxes, logits, scale_factors)
